# baseline (speedup 1.0000x reference)
import jax
import jax.numpy as jnp
from jax.experimental import pallas as pl

_B, _S, _D, _K, _TOPK = 2, 2048, 1024, 128, 64
_BM = 1024
_BQ = 512
_NEG = -1e9


def _proj_kernel(x_ref, p_ref, wqt_t_ref, wkt_t_ref, wq_t_ref, wk_t_ref,
                 wv_t_ref, tq_ref, tk_ref, q_ref, km_ref, v_ref):
    x = x_ref[...]
    xp = jnp.dot(x, p_ref[...], preferred_element_type=jnp.float32)
    tq_ref[...] = jnp.dot(xp, wqt_t_ref[...], preferred_element_type=jnp.float32)
    tk_ref[...] = jnp.dot(xp, wkt_t_ref[...], preferred_element_type=jnp.float32)
    q_ref[...] = jnp.dot(x, wq_t_ref[...], preferred_element_type=jnp.float32)
    km_ref[...] = jnp.dot(x, wk_t_ref[...], preferred_element_type=jnp.float32)
    v_ref[...] = jnp.dot(x, wv_t_ref[...], preferred_element_type=jnp.float32)


def _sortable(x):
    k = jax.lax.bitcast_convert_type(x, jnp.int32)
    return k ^ (jax.lax.shift_right_arithmetic(k, 31) & jnp.int32(0x7FFFFFFF))


def _attn_kernel(tq_ref, tk_ref, q_ref, km_ref, v_ref, o_ref):
    ts = jax.lax.dot_general(
        tq_ref[0], tk_ref[0], (((1,), (1,)), ((), ())),
        preferred_element_type=jnp.float32)

    sm = jax.lax.dot_general(
        q_ref[0], km_ref[0], (((1,), (1,)), ((), ())),
        preferred_element_type=jnp.float32)

    sign = jnp.int32(-(2 ** 31))
    rows = ts.shape[0]

    m16 = ts[:, 0:128]
    for c in range(1, 16):
        m16 = jnp.maximum(m16, ts[:, c * 128:(c + 1) * 128])
    ub = jnp.max(m16, axis=-1, keepdims=True)
    lb = jnp.min(m16, axis=-1, keepdims=True)
    kub, klb = _sortable(ub), _sortable(lb)
    pub = kub ^ sign
    d = kub ^ klb
    df = d.astype(jnp.float32)
    b0 = jnp.where(
        d < 0,
        jnp.int32(31),
        jnp.maximum(
            (jax.lax.shift_right_logical(
                jax.lax.bitcast_convert_type(df, jnp.int32), 23) & 0xFF)
            - 127, 0).astype(jnp.int32))
    lowmask = ~jax.lax.shift_left(jnp.int32(-2), b0)
    u0 = pub & ~lowmask

    def cond(state):
        i, _, cnt = state
        return jnp.logical_and(i < 32, jnp.any(cnt != jnp.float32(_TOPK)))

    def _unsort(p):
        y = p ^ sign
        fb = jnp.where(y >= 0, y, y ^ jnp.int32(0x7FFFFFFF))
        return jax.lax.bitcast_convert_type(fb, jnp.float32)

    topkf = jnp.float32(_TOPK)

    def _count(fthr):
        accs = [None] * 2
        for c in range(16):
            ind = jnp.where(ts[:, c * 128:(c + 1) * 128] >= fthr, 1.0, 0.0)
            accs[c % 2] = ind if accs[c % 2] is None else accs[c % 2] + ind
        return jnp.sum(accs[0] + accs[1], axis=-1, keepdims=True)

    def _step(i, u, cnt):
        bit = jax.lax.shift_left(jnp.int32(1), jnp.maximum(b0 - i, 0))
        cand = u | bit
        c = _count(_unsort(cand))
        ok = c >= topkf
        u = jnp.where(ok, cand, u)
        cnt = jnp.where(ok, c, cnt)
        return u, cnt

    def body(state):
        i, u, cnt = state
        u, cnt = _step(i, u, cnt)
        u, cnt = _step(i + 1, u, cnt)
        return i + 2, u, cnt

    _, u, _ = jax.lax.while_loop(
        cond, body,
        (jnp.int32(0), u0, jnp.full((rows, 1), 2.0 * _TOPK, jnp.float32)))
    sm = jnp.where(ts >= _unsort(u), sm, _NEG)
    m = jnp.max(sm, axis=-1, keepdims=True)
    e = jnp.exp(sm - m)
    a = e / jnp.sum(e, axis=-1, keepdims=True)
    o_ref[0] = jnp.dot(a, v_ref[0], preferred_element_type=jnp.float32)


def _projections(Xf, P, WqtT, WktT, WqT, WkT, WvT, interpret=False):
    n = _B * _S
    grid = (n // _BM,)
    row = lambda i: (i, 0)
    fixed = lambda i: (0, 0)
    return pl.pallas_call(
        _proj_kernel,
        grid=grid,
        in_specs=[
            pl.BlockSpec((_BM, _D), row),
            pl.BlockSpec((_D, _K), fixed),
            pl.BlockSpec((_K, _K), fixed),
            pl.BlockSpec((_K, _K), fixed),
            pl.BlockSpec((_D, _D), fixed),
            pl.BlockSpec((_D, _D), fixed),
            pl.BlockSpec((_D, _D), fixed),
        ],
        out_specs=[
            pl.BlockSpec((_BM, _K), row),
            pl.BlockSpec((_BM, _K), row),
            pl.BlockSpec((_BM, _D), row),
            pl.BlockSpec((_BM, _D), row),
            pl.BlockSpec((_BM, _D), row),
        ],
        out_shape=[
            jax.ShapeDtypeStruct((n, _K), jnp.float32),
            jax.ShapeDtypeStruct((n, _K), jnp.float32),
            jax.ShapeDtypeStruct((n, _D), jnp.float32),
            jax.ShapeDtypeStruct((n, _D), jnp.float32),
            jax.ShapeDtypeStruct((n, _D), jnp.float32),
        ],
        interpret=interpret,
    )(Xf, P, WqtT, WktT, WqT, WkT, WvT)


def _attention(tq, tk, q, km, v, interpret=False):
    qblk = lambda b, i: (b, i, 0)
    kall = lambda b, i: (b, 0, 0)
    return pl.pallas_call(
        _attn_kernel,
        grid=(_B, _S // _BQ),
        in_specs=[
            pl.BlockSpec((1, _BQ, _K), qblk),
            pl.BlockSpec((1, _S, _K), kall),
            pl.BlockSpec((1, _BQ, _D), qblk),
            pl.BlockSpec((1, _S, _D), kall),
            pl.BlockSpec((1, _S, _D), kall),
        ],
        out_specs=pl.BlockSpec((1, _BQ, _D), qblk),
        out_shape=jax.ShapeDtypeStruct((_B, _S, _D), jnp.float32),
        interpret=interpret,
    )(tq, tk, q, km, v)


def kernel(X, P, Wq_tilde, Wk_tilde, Wq, Wk, Wv, interpret=False):
    Xf = X.reshape(_B * _S, _D)
    tq, tk, q, km, v = _projections(
        Xf, P, Wq_tilde.T, Wk_tilde.T, Wq.T, Wk.T, Wv.T, interpret=interpret)
    r3 = lambda t, w: t.reshape(_B, _S, w)
    return _attention(r3(tq, _K), r3(tk, _K), r3(q, _D), r3(km, _D),
                      r3(v, _D), interpret=interpret)


# speedup vs baseline: 1.0037x; 1.0037x over previous
import jax
import jax.numpy as jnp
from jax.experimental import pallas as pl

_B, _S, _D, _K, _TOPK = 2, 2048, 1024, 128, 64
_BM = 512
_BQ = 512
_NEG = -1e9


def _proj_kernel(x_ref, p_ref, wqt_t_ref, wkt_t_ref, wq_t_ref, wk_t_ref,
                 wv_t_ref, tq_ref, tk_ref, q_ref, km_ref, v_ref):
    x = x_ref[...]
    xp = jnp.dot(x, p_ref[...], preferred_element_type=jnp.float32)
    tq_ref[...] = jnp.dot(xp, wqt_t_ref[...], preferred_element_type=jnp.float32)
    tk_ref[...] = jnp.dot(xp, wkt_t_ref[...], preferred_element_type=jnp.float32)
    q_ref[...] = jnp.dot(x, wq_t_ref[...], preferred_element_type=jnp.float32)
    km_ref[...] = jnp.dot(x, wk_t_ref[...], preferred_element_type=jnp.float32)
    v_ref[...] = jnp.dot(x, wv_t_ref[...], preferred_element_type=jnp.float32)


def _sortable(x):
    k = jax.lax.bitcast_convert_type(x, jnp.int32)
    return k ^ (jax.lax.shift_right_arithmetic(k, 31) & jnp.int32(0x7FFFFFFF))


def _attn_kernel(tq_ref, tk_ref, q_ref, km_ref, v_ref, o_ref):
    ts = jax.lax.dot_general(
        tq_ref[0], tk_ref[0], (((1,), (1,)), ((), ())),
        preferred_element_type=jnp.float32)

    sm = jax.lax.dot_general(
        q_ref[0], km_ref[0], (((1,), (1,)), ((), ())),
        preferred_element_type=jnp.float32)

    sign = jnp.int32(-(2 ** 31))
    rows = ts.shape[0]

    m16 = ts[:, 0:128]
    for c in range(1, 16):
        m16 = jnp.maximum(m16, ts[:, c * 128:(c + 1) * 128])
    ub = jnp.max(m16, axis=-1, keepdims=True)
    lb = jnp.min(m16, axis=-1, keepdims=True)
    kub, klb = _sortable(ub), _sortable(lb)
    pub = kub ^ sign
    d = kub ^ klb
    df = d.astype(jnp.float32)
    b0 = jnp.where(
        d < 0,
        jnp.int32(31),
        jnp.maximum(
            (jax.lax.shift_right_logical(
                jax.lax.bitcast_convert_type(df, jnp.int32), 23) & 0xFF)
            - 127, 0).astype(jnp.int32))
    lowmask = ~jax.lax.shift_left(jnp.int32(-2), b0)
    u0 = pub & ~lowmask

    def cond(state):
        i, _, cnt = state
        return jnp.logical_and(i < 32, jnp.any(cnt != jnp.float32(_TOPK)))

    def _unsort(p):
        y = p ^ sign
        fb = jnp.where(y >= 0, y, y ^ jnp.int32(0x7FFFFFFF))
        return jax.lax.bitcast_convert_type(fb, jnp.float32)

    topkf = jnp.float32(_TOPK)

    def _count(fthr):
        accs = [None] * 2
        for c in range(16):
            ind = jnp.where(ts[:, c * 128:(c + 1) * 128] >= fthr, 1.0, 0.0)
            accs[c % 2] = ind if accs[c % 2] is None else accs[c % 2] + ind
        return jnp.sum(accs[0] + accs[1], axis=-1, keepdims=True)

    def _step(i, u, cnt):
        bit = jax.lax.shift_left(jnp.int32(1), jnp.maximum(b0 - i, 0))
        cand = u | bit
        c = _count(_unsort(cand))
        ok = c >= topkf
        u = jnp.where(ok, cand, u)
        cnt = jnp.where(ok, c, cnt)
        return u, cnt

    def body(state):
        i, u, cnt = state
        u, cnt = _step(i, u, cnt)
        u, cnt = _step(i + 1, u, cnt)
        return i + 2, u, cnt

    _, u, _ = jax.lax.while_loop(
        cond, body,
        (jnp.int32(0), u0, jnp.full((rows, 1), 2.0 * _TOPK, jnp.float32)))
    sm = jnp.where(ts >= _unsort(u), sm, _NEG)
    m = jnp.max(sm, axis=-1, keepdims=True)
    e = jnp.exp(sm - m)
    a = e / jnp.sum(e, axis=-1, keepdims=True)
    o_ref[0] = jnp.dot(a, v_ref[0], preferred_element_type=jnp.float32)


def _projections(Xf, P, WqtT, WktT, WqT, WkT, WvT, interpret=False):
    n = _B * _S
    grid = (n // _BM,)
    row = lambda i: (i, 0)
    fixed = lambda i: (0, 0)
    return pl.pallas_call(
        _proj_kernel,
        grid=grid,
        in_specs=[
            pl.BlockSpec((_BM, _D), row),
            pl.BlockSpec((_D, _K), fixed),
            pl.BlockSpec((_K, _K), fixed),
            pl.BlockSpec((_K, _K), fixed),
            pl.BlockSpec((_D, _D), fixed),
            pl.BlockSpec((_D, _D), fixed),
            pl.BlockSpec((_D, _D), fixed),
        ],
        out_specs=[
            pl.BlockSpec((_BM, _K), row),
            pl.BlockSpec((_BM, _K), row),
            pl.BlockSpec((_BM, _D), row),
            pl.BlockSpec((_BM, _D), row),
            pl.BlockSpec((_BM, _D), row),
        ],
        out_shape=[
            jax.ShapeDtypeStruct((n, _K), jnp.float32),
            jax.ShapeDtypeStruct((n, _K), jnp.float32),
            jax.ShapeDtypeStruct((n, _D), jnp.float32),
            jax.ShapeDtypeStruct((n, _D), jnp.float32),
            jax.ShapeDtypeStruct((n, _D), jnp.float32),
        ],
        interpret=interpret,
    )(Xf, P, WqtT, WktT, WqT, WkT, WvT)


def _attention(tq, tk, q, km, v, interpret=False):
    qblk = lambda b, i: (b, i, 0)
    kall = lambda b, i: (b, 0, 0)
    return pl.pallas_call(
        _attn_kernel,
        grid=(_B, _S // _BQ),
        in_specs=[
            pl.BlockSpec((1, _BQ, _K), qblk),
            pl.BlockSpec((1, _S, _K), kall),
            pl.BlockSpec((1, _BQ, _D), qblk),
            pl.BlockSpec((1, _S, _D), kall),
            pl.BlockSpec((1, _S, _D), kall),
        ],
        out_specs=pl.BlockSpec((1, _BQ, _D), qblk),
        out_shape=jax.ShapeDtypeStruct((_B, _S, _D), jnp.float32),
        interpret=interpret,
    )(tq, tk, q, km, v)


def kernel(X, P, Wq_tilde, Wk_tilde, Wq, Wk, Wv, interpret=False):
    Xf = X.reshape(_B * _S, _D)
    tq, tk, q, km, v = _projections(
        Xf, P, Wq_tilde.T, Wk_tilde.T, Wq.T, Wk.T, Wv.T, interpret=interpret)
    r3 = lambda t, w: t.reshape(_B, _S, w)
    return _attention(r3(tq, _K), r3(tk, _K), r3(q, _D), r3(km, _D),
                      r3(v, _D), interpret=interpret)
